# baseline (speedup 1.0000x reference)
import jax
import jax.numpy as jnp
from jax.experimental import pallas as pl

_NB, _NA, _ATTR = 16, 3, 85
_NH = _NW = 76
_NP = _NH * _NW
_NC = _NA * _ATTR
_STRIDE = 8.0
_AW = (116.0, 156.0, 373.0)
_AH = (90.0, 198.0, 326.0)

_TH = 16
_NT = (_NH + _TH - 1) // _TH
_TP = _TH * _NW
_TR = _TP * _NA


def _body(x_ref, o_ref):
    j = pl.program_id(1)
    v = x_ref[0].reshape(_NC, _TP)
    t = v.T
    c = jax.lax.broadcasted_iota(jnp.int32, (1, _NC), 1)
    r = c % _ATTR
    a = c // _ATTR
    isexp = (r == 2) | (r == 3)
    e = jnp.exp(jnp.where(isexp, t, -t))
    base = jnp.where(isexp, e, 1.0 / (1.0 + e))
    aw = jnp.where(a == 0, _AW[0], jnp.where(a == 1, _AW[1], _AW[2]))
    ah = jnp.where(a == 0, _AH[0], jnp.where(a == 1, _AH[1], _AH[2]))
    mul = jnp.where(r < 2, _STRIDE,
          jnp.where(r == 2, aw,
          jnp.where(r == 3, ah, 1.0))).astype(jnp.float32)
    p = j * _TP + jax.lax.broadcasted_iota(jnp.int32, (_TP, 1), 0)
    w = (p % _NW).astype(jnp.float32)
    h = (p // _NW).astype(jnp.float32)
    m0 = (r == 0).astype(jnp.float32)
    m1 = (r == 1).astype(jnp.float32)
    add = m0 * (_STRIDE * w) + m1 * (_STRIDE * h)
    res = base * mul + add
    for anc in range(_NA):
        o_ref[0, pl.Slice(anc, _TP, _NA), :] = res[:, anc * _ATTR:(anc + 1) * _ATTR]


def kernel(x):
    return pl.pallas_call(
        _body,
        grid=(_NB, _NT),
        in_specs=[pl.BlockSpec((1, _NC, _TH, _NW), lambda b, j: (b, 0, j, 0))],
        out_specs=pl.BlockSpec((1, _TR, _ATTR), lambda b, j: (b, j, 0)),
        out_shape=jax.ShapeDtypeStruct((_NB, _NP * _NA, _ATTR), jnp.float32),
    )(x)


# speedup vs baseline: 2.0126x; 2.0126x over previous
import jax
import jax.numpy as jnp
from jax.experimental import pallas as pl

_NB, _NA, _ATTR = 16, 3, 85
_NH = _NW = 76
_NP = _NH * _NW
_NC = _NA * _ATTR
_STRIDE = 8.0
_AW = (116.0, 156.0, 373.0)
_AH = (90.0, 198.0, 326.0)

_TPP = 512
_NT = (_NP + _TPP - 1) // _TPP
_TR = _TPP * _NA


def _body(x_ref, o_ref):
    j = pl.program_id(0)
    v = x_ref[...]
    w = jnp.transpose(v, (1, 0, 2))
    c = jax.lax.broadcasted_iota(jnp.int32, (1, _NC), 1)
    r = c % _ATTR
    a = c // _ATTR
    isexp = (r == 2) | (r == 3)
    aw = jnp.where(a == 0, _AW[0], jnp.where(a == 1, _AW[1], _AW[2]))
    ah = jnp.where(a == 0, _AH[0], jnp.where(a == 1, _AH[1], _AH[2]))
    mul = jnp.where(r < 2, _STRIDE,
          jnp.where(r == 2, aw,
          jnp.where(r == 3, ah, 1.0))).astype(jnp.float32)
    m0 = (r == 0).astype(jnp.float32)
    m1 = (r == 1).astype(jnp.float32)
    p = j * _TPP + jax.lax.broadcasted_iota(jnp.int32, (_TPP, 1), 0)
    wm = (p % _NW).astype(jnp.float32)
    hm = (p // _NW).astype(jnp.float32)
    add = m0 * (_STRIDE * wm) + m1 * (_STRIDE * hm)
    for b in range(_NB):
        t = w[b]
        e = jnp.exp(jnp.where(isexp, t, -t))
        base = jnp.where(isexp, e, 1.0 / (1.0 + e))
        res = base * mul + add
        for anc in range(_NA):
            o_ref[b, pl.Slice(anc, _TPP, _NA), :] = (
                res[:, anc * _ATTR:(anc + 1) * _ATTR])


def kernel(x):
    xt = jnp.transpose(x, (2, 3, 0, 1)).reshape(_NP, _NB, _NC)
    return pl.pallas_call(
        _body,
        grid=(_NT,),
        in_specs=[pl.BlockSpec((_TPP, _NB, _NC), lambda j: (j, 0, 0))],
        out_specs=pl.BlockSpec((_NB, _TR, _ATTR), lambda j: (0, j, 0)),
        out_shape=jax.ShapeDtypeStruct((_NB, _NP * _NA, _ATTR), jnp.float32),
    )(xt)
